# baseline (speedup 1.0000x reference)
import functools

import jax
import jax.numpy as jnp
from jax import lax
from jax.experimental import pallas as pl
from jax.experimental.pallas import tpu as pltpu
from jax.experimental.pallas import tpu_sc as plsc

BATCH = 16384
EMBED_DIM = 128
CHUNK = 128

_NC, _NS = 2, 16
_NW = _NC * _NS
_B_PER_W = BATCH // _NW
_CH_PER_W = _B_PER_W // CHUNK


def _sc_gather_body(users_hbm, items_hbm, ut_hbm, it_hbm, ue_hbm, ie_hbm,
                    idx_v, rows_v, sem):
    wid = lax.axis_index("s") * _NC + lax.axis_index("c")
    r0 = wid * _CH_PER_W
    for src, tbl, dst in ((users_hbm, ut_hbm, ue_hbm),
                          (items_hbm, it_hbm, ie_hbm)):
        pltpu.sync_copy(src.at[pl.ds(r0, _CH_PER_W)], idx_v)
        copies = []
        for j in range(_CH_PER_W):
            copies.append(pltpu.async_copy(
                tbl.at[idx_v.at[j]], rows_v.at[pl.ds(j * CHUNK, CHUNK)], sem))
        for c in copies:
            c.wait()
        pltpu.sync_copy(rows_v, dst.at[pl.ds(r0 * CHUNK, _B_PER_W)])


def _sc_gather(users_r, items_r, user_table, item_table):
    mesh = plsc.VectorSubcoreMesh(core_axis_name="c", subcore_axis_name="s")
    f = pl.kernel(
        _sc_gather_body,
        mesh=mesh,
        out_type=(
            jax.ShapeDtypeStruct((BATCH, EMBED_DIM), jnp.float32),
            jax.ShapeDtypeStruct((BATCH, EMBED_DIM), jnp.float32),
        ),
        scratch_types=[
            pltpu.VMEM((_CH_PER_W, CHUNK), jnp.int32),
            pltpu.VMEM((_B_PER_W, EMBED_DIM), jnp.float32),
            pltpu.SemaphoreType.DMA,
        ],
    )
    return f(users_r, items_r, user_table, item_table)


def _mlp_body(ue_ref, ie_ref, w0_ref, b0_ref, w1_ref, b1_ref, w2_ref, b2_ref,
              out_ref):
    bf = jnp.bfloat16
    h = jnp.dot(ue_ref[...].astype(bf), w0_ref[0:EMBED_DIM, :],
                preferred_element_type=jnp.float32)
    h += jnp.dot(ie_ref[...].astype(bf), w0_ref[EMBED_DIM:2 * EMBED_DIM, :],
                 preferred_element_type=jnp.float32)
    h = jnp.maximum(h + b0_ref[...], 0.0)
    h = jnp.dot(h.astype(bf), w1_ref[...], preferred_element_type=jnp.float32)
    h = jnp.maximum(h + b1_ref[...], 0.0)
    h = jnp.dot(h.astype(bf), w2_ref[...], preferred_element_type=jnp.float32)
    out_ref[...] = jnp.maximum(h + b2_ref[...], 0.0)


def _mlp(ue, ie, W0, b0, W1, b1, W2, b2, blk=1024):
    n_blk = BATCH // blk
    h0, h1, h2 = W0.shape[1], W1.shape[1], W2.shape[1]
    return pl.pallas_call(
        _mlp_body,
        grid=(n_blk,),
        in_specs=[
            pl.BlockSpec((blk, EMBED_DIM), lambda i: (i, 0)),
            pl.BlockSpec((blk, EMBED_DIM), lambda i: (i, 0)),
            pl.BlockSpec((2 * EMBED_DIM, h0), lambda i: (0, 0)),
            pl.BlockSpec((1, h0), lambda i: (0, 0)),
            pl.BlockSpec((h0, h1), lambda i: (0, 0)),
            pl.BlockSpec((1, h1), lambda i: (0, 0)),
            pl.BlockSpec((h1, h2), lambda i: (0, 0)),
            pl.BlockSpec((1, h2), lambda i: (0, 0)),
        ],
        out_specs=pl.BlockSpec((blk, h2), lambda i: (i, 0)),
        out_shape=jax.ShapeDtypeStruct((BATCH, h2), jnp.float32),
        compiler_params=pltpu.CompilerParams(
            dimension_semantics=("arbitrary",),
        ),
    )(ue, ie, W0.astype(jnp.bfloat16), b0.reshape(1, h0),
      W1.astype(jnp.bfloat16), b1.reshape(1, h1),
      W2.astype(jnp.bfloat16), b2.reshape(1, h2))


def kernel(users, items, user_table, item_table, W0, b0, W1, b1, W2, b2):
    users_r = users.astype(jnp.int32).reshape(BATCH // CHUNK, CHUNK)
    items_r = items.astype(jnp.int32).reshape(BATCH // CHUNK, CHUNK)
    ue, ie = _sc_gather(users_r, items_r, user_table, item_table)
    return _mlp(ue, ie, W0, b0, W1, b1, W2, b2)


# speedup vs baseline: 1.1440x; 1.1440x over previous
import jax
import jax.numpy as jnp
from jax import lax
from jax.experimental import pallas as pl
from jax.experimental.pallas import tpu as pltpu
from jax.experimental.pallas import tpu_sc as plsc

BATCH = 16384
EMBED_DIM = 128
CHUNK = 128

_NC, _NS = 2, 16
_NW = _NC * _NS
_B_PER_W = BATCH // _NW
_CH_PER_W = _B_PER_W // CHUNK


def _sc_gather_body(users_hbm, items_hbm, ut_hbm, it_hbm, x_hbm,
                    idx_v, rows_v, sem):
    wid = lax.axis_index("s") * _NC + lax.axis_index("c")
    r0 = wid * _CH_PER_W
    for col, (src, tbl) in enumerate(((users_hbm, ut_hbm), (items_hbm, it_hbm))):
        pltpu.sync_copy(src.at[pl.ds(r0, _CH_PER_W)], idx_v)
        copies = []
        for j in range(_CH_PER_W):
            copies.append(pltpu.async_copy(
                tbl.at[idx_v.at[j]], rows_v.at[pl.ds(j * CHUNK, CHUNK)], sem))
        for c in copies:
            c.wait()
        pltpu.sync_copy(
            rows_v,
            x_hbm.at[pl.ds(r0 * CHUNK, _B_PER_W),
                     pl.ds(col * EMBED_DIM, EMBED_DIM)])


def _sc_gather(users_r, items_r, user_table, item_table):
    mesh = plsc.VectorSubcoreMesh(core_axis_name="c", subcore_axis_name="s")
    f = pl.kernel(
        _sc_gather_body,
        mesh=mesh,
        out_type=jax.ShapeDtypeStruct((BATCH, 2 * EMBED_DIM), jnp.float32),
        scratch_types=[
            pltpu.VMEM((_CH_PER_W, CHUNK), jnp.int32),
            pltpu.VMEM((_B_PER_W, EMBED_DIM), jnp.float32),
            pltpu.SemaphoreType.DMA,
        ],
    )
    return f(users_r, items_r, user_table, item_table)


def _mlp_body(x_ref, w0_ref, b0_ref, w1_ref, b1_ref, w2_ref, b2_ref, out_ref):
    bf = jnp.bfloat16
    h = jnp.dot(x_ref[...].astype(bf), w0_ref[...],
                preferred_element_type=jnp.float32)
    h = jnp.maximum(h + b0_ref[...], 0.0)
    h = jnp.dot(h.astype(bf), w1_ref[...], preferred_element_type=jnp.float32)
    h = jnp.maximum(h + b1_ref[...], 0.0)
    h = jnp.dot(h.astype(bf), w2_ref[...], preferred_element_type=jnp.float32)
    out_ref[...] = jnp.maximum(h + b2_ref[...], 0.0)


def _mlp(x, W0, b0, W1, b1, W2, b2, blk=2048):
    n_blk = BATCH // blk
    h0, h1, h2 = W0.shape[1], W1.shape[1], W2.shape[1]
    return pl.pallas_call(
        _mlp_body,
        grid=(n_blk,),
        in_specs=[
            pl.BlockSpec((blk, 2 * EMBED_DIM), lambda i: (i, 0)),
            pl.BlockSpec((2 * EMBED_DIM, h0), lambda i: (0, 0)),
            pl.BlockSpec((1, h0), lambda i: (0, 0)),
            pl.BlockSpec((h0, h1), lambda i: (0, 0)),
            pl.BlockSpec((1, h1), lambda i: (0, 0)),
            pl.BlockSpec((h1, h2), lambda i: (0, 0)),
            pl.BlockSpec((1, h2), lambda i: (0, 0)),
        ],
        out_specs=pl.BlockSpec((blk, h2), lambda i: (i, 0)),
        out_shape=jax.ShapeDtypeStruct((BATCH, h2), jnp.float32),
        compiler_params=pltpu.CompilerParams(
            dimension_semantics=("arbitrary",),
        ),
    )(x, W0.astype(jnp.bfloat16), b0.reshape(1, h0),
      W1.astype(jnp.bfloat16), b1.reshape(1, h1),
      W2.astype(jnp.bfloat16), b2.reshape(1, h2))


def kernel(users, items, user_table, item_table, W0, b0, W1, b1, W2, b2):
    users_r = users.astype(jnp.int32).reshape(BATCH // CHUNK, CHUNK)
    items_r = items.astype(jnp.int32).reshape(BATCH // CHUNK, CHUNK)
    x = _sc_gather(users_r, items_r, user_table, item_table)
    return _mlp(x, W0, b0, W1, b1, W2, b2)


# speedup vs baseline: 1.1509x; 1.0060x over previous
import jax
import jax.numpy as jnp
from jax import lax
from jax.experimental import pallas as pl
from jax.experimental.pallas import tpu as pltpu
from jax.experimental.pallas import tpu_sc as plsc

BATCH = 16384
EMBED_DIM = 128
CHUNK = 128

_NC, _NS = 2, 16
_NW = _NC * _NS
_B_PER_W = BATCH // _NW
_CH_PER_W = _B_PER_W // CHUNK


def _sc_gather_body(users_hbm, items_hbm, ut_hbm, it_hbm, x_hbm,
                    idx_v, rows_v, sem):
    wid = lax.axis_index("s") * _NC + lax.axis_index("c")
    r0 = wid * _CH_PER_W
    for col, (src, tbl) in enumerate(((users_hbm, ut_hbm), (items_hbm, it_hbm))):
        pltpu.sync_copy(src.at[pl.ds(r0, _CH_PER_W)], idx_v)
        copies = []
        for j in range(_CH_PER_W):
            copies.append(pltpu.async_copy(
                tbl.at[idx_v.at[j]], rows_v.at[pl.ds(j * CHUNK, CHUNK)], sem))
        for c in copies:
            c.wait()
        pltpu.sync_copy(
            rows_v,
            x_hbm.at[pl.ds(r0 * CHUNK, _B_PER_W),
                     pl.ds(col * EMBED_DIM, EMBED_DIM)])


def _sc_gather(users_r, items_r, user_table, item_table):
    mesh = plsc.VectorSubcoreMesh(core_axis_name="c", subcore_axis_name="s")
    f = pl.kernel(
        _sc_gather_body,
        mesh=mesh,
        out_type=jax.ShapeDtypeStruct((BATCH, 2 * EMBED_DIM), jnp.float32),
        scratch_types=[
            pltpu.VMEM((_CH_PER_W, CHUNK), jnp.int32),
            pltpu.VMEM((_B_PER_W, EMBED_DIM), jnp.float32),
            pltpu.SemaphoreType.DMA,
        ],
    )
    return f(users_r, items_r, user_table, item_table)


def _mlp_body(x_ref, w0_ref, b0_ref, w1_ref, b1_ref, w2_ref, b2_ref, out_ref):
    bf = jnp.bfloat16
    h = jnp.dot(x_ref[...].astype(bf), w0_ref[...],
                preferred_element_type=jnp.float32)
    h = jnp.maximum(h + b0_ref[...], 0.0)
    h = jnp.dot(h.astype(bf), w1_ref[...], preferred_element_type=jnp.float32)
    h = jnp.maximum(h + b1_ref[...], 0.0)
    h = jnp.dot(h.astype(bf), w2_ref[...], preferred_element_type=jnp.float32)
    out_ref[...] = jnp.maximum(h + b2_ref[...], 0.0)


def _mlp(x, W0, b0, W1, b1, W2, b2, blk=4096):
    n_blk = BATCH // blk
    h0, h1, h2 = W0.shape[1], W1.shape[1], W2.shape[1]
    return pl.pallas_call(
        _mlp_body,
        grid=(n_blk,),
        in_specs=[
            pl.BlockSpec((blk, 2 * EMBED_DIM), lambda i: (i, 0)),
            pl.BlockSpec((2 * EMBED_DIM, h0), lambda i: (0, 0)),
            pl.BlockSpec((1, h0), lambda i: (0, 0)),
            pl.BlockSpec((h0, h1), lambda i: (0, 0)),
            pl.BlockSpec((1, h1), lambda i: (0, 0)),
            pl.BlockSpec((h1, h2), lambda i: (0, 0)),
            pl.BlockSpec((1, h2), lambda i: (0, 0)),
        ],
        out_specs=pl.BlockSpec((blk, h2), lambda i: (i, 0)),
        out_shape=jax.ShapeDtypeStruct((BATCH, h2), jnp.float32),
        compiler_params=pltpu.CompilerParams(
            dimension_semantics=("arbitrary",),
        ),
    )(x, W0.astype(jnp.bfloat16), b0.reshape(1, h0),
      W1.astype(jnp.bfloat16), b1.reshape(1, h1),
      W2.astype(jnp.bfloat16), b2.reshape(1, h2))


def kernel(users, items, user_table, item_table, W0, b0, W1, b1, W2, b2):
    users_r = users.astype(jnp.int32).reshape(BATCH // CHUNK, CHUNK)
    items_r = items.astype(jnp.int32).reshape(BATCH // CHUNK, CHUNK)
    x = _sc_gather(users_r, items_r, user_table, item_table)
    return _mlp(x, W0, b0, W1, b1, W2, b2)
